# baseline (speedup 1.0000x reference)
import functools

import jax
import jax.numpy as jnp
from jax import lax
from jax.experimental import pallas as pl
from jax.experimental.pallas import tpu as pltpu
from jax.experimental.pallas import tpu_sc as plsc

N_NODES = 10000
N_EDGES = 160000
IN_CH = 256
HIDDEN = 512
OUT_CH = 128

NC = 2
NS = 16
NW = NC * NS
CHUNK = 128
CHUNKS_PER_TILE = 40
N_CHUNKS_PAD = NW * CHUNKS_PER_TILE
ACC_ROWS = N_NODES + 16
DUMP_ROW = N_NODES
ROWS_A = 624
ROWS_LAST = N_NODES - (NS - 1) * ROWS_A


def _mlp_body(x_ref, w1_ref, w2_ref, h_ref):
    t = jnp.dot(x_ref[...], w1_ref[...], preferred_element_type=jnp.float32)
    t = jnp.maximum(t, 0.0)
    h_ref[...] = jnp.dot(t, w2_ref[...], preferred_element_type=jnp.float32)


def _mlp(x, W1, W2):
    R = 1000
    return pl.pallas_call(
        _mlp_body,
        grid=(N_NODES // R,),
        in_specs=[
            pl.BlockSpec((R, IN_CH), lambda i: (i, 0)),
            pl.BlockSpec((IN_CH, HIDDEN), lambda i: (0, 0)),
            pl.BlockSpec((HIDDEN, OUT_CH), lambda i: (0, 0)),
        ],
        out_specs=pl.BlockSpec((R, OUT_CH), lambda i: (i, 0)),
        out_shape=jax.ShapeDtypeStruct((N_NODES, OUT_CH), jnp.float32),
    )(x, W1, W2)


def _sc_body(h_hbm, src_hbm, dst_hbm, out_hbm,
             acc, srcbuf, dstbuf, rows, sem_a):
    cid = lax.axis_index("c")
    sid = lax.axis_index("s")
    wid = sid * NC + cid
    rbase = pl.multiple_of(sid * ROWS_A, 8)

    cbase = pl.multiple_of(wid * CHUNKS_PER_TILE, 8)
    pltpu.sync_copy(src_hbm.at[pl.ds(cbase, CHUNKS_PER_TILE)], srcbuf)
    pltpu.sync_copy(dst_hbm.at[pl.ds(cbase, CHUNKS_PER_TILE)], dstbuf)

    def zrow(i, carry):
        def zcol(j, c2):
            rows[0, i, pl.ds(j * 16, 16)] = jnp.zeros((16,), jnp.float32)
            return c2
        return lax.fori_loop(0, OUT_CH // 16, zcol, carry)
    lax.fori_loop(0, CHUNK, zrow, 0)

    @pl.when(sid < NS - 1)
    def _():
        for k in range(ROWS_A // CHUNK):
            pltpu.sync_copy(rows.at[0], acc.at[pl.ds(rbase + k * CHUNK, CHUNK)])
        tail = ROWS_A % CHUNK
        pltpu.sync_copy(rows.at[0].at[pl.ds(0, tail)],
                        acc.at[pl.ds(rbase + ROWS_A - tail, tail)])

    @pl.when(sid == NS - 1)
    def _():
        for k in range(ROWS_LAST // CHUNK):
            pltpu.sync_copy(rows.at[0], acc.at[pl.ds(rbase + k * CHUNK, CHUNK)])
    plsc.subcore_barrier()

    def step(i, carry):
        pltpu.async_copy(h_hbm.at[srcbuf.at[i]], rows.at[0], sem_a).wait()
        pltpu.sync_copy(rows.at[0], acc.at[dstbuf.at[i]], add=True)
        return carry
    lax.fori_loop(0, CHUNKS_PER_TILE, step, 0)

    plsc.subcore_barrier()

    @pl.when(sid < NS - 1)
    def _():
        for k in range(ROWS_A // CHUNK):
            sl = pl.ds(rbase + k * CHUNK, CHUNK)
            pltpu.sync_copy(acc.at[sl], rows.at[0])
            pltpu.sync_copy(rows.at[0], out_hbm.at[cid, sl])
        tail = ROWS_A % CHUNK
        sl = pl.ds(rbase + ROWS_A - tail, tail)
        pltpu.sync_copy(acc.at[sl], rows.at[0].at[pl.ds(0, tail)])
        pltpu.sync_copy(rows.at[0].at[pl.ds(0, tail)], out_hbm.at[cid, sl])

    @pl.when(sid == NS - 1)
    def _():
        for k in range(ROWS_LAST // CHUNK):
            sl = pl.ds(rbase + k * CHUNK, CHUNK)
            pltpu.sync_copy(acc.at[sl], rows.at[0])
            pltpu.sync_copy(rows.at[0], out_hbm.at[cid, sl])


def _sc_scatter(h, src2d, dst2d):
    mesh = plsc.VectorSubcoreMesh(core_axis_name="c", subcore_axis_name="s")
    fn = pl.kernel(
        _sc_body,
        out_type=jax.ShapeDtypeStruct((NC, N_NODES, OUT_CH), jnp.float32),
        mesh=mesh,
        scratch_types=[
            pltpu.VMEM_SHARED((ACC_ROWS, OUT_CH), jnp.float32),
            pltpu.VMEM((CHUNKS_PER_TILE, CHUNK), jnp.int32),
            pltpu.VMEM((CHUNKS_PER_TILE, CHUNK), jnp.int32),
            pltpu.VMEM((2, CHUNK, OUT_CH), jnp.float32),
            pltpu.SemaphoreType.DMA,
        ],
    )
    return fn(h, src2d, dst2d)


def _add_body(p_ref, o_ref):
    o_ref[...] = p_ref[0] + p_ref[1]


def _combine(partials):
    R = 2000
    return pl.pallas_call(
        _add_body,
        grid=(N_NODES // R,),
        in_specs=[pl.BlockSpec((NC, R, OUT_CH), lambda i: (0, i, 0))],
        out_specs=pl.BlockSpec((R, OUT_CH), lambda i: (i, 0)),
        out_shape=jax.ShapeDtypeStruct((N_NODES, OUT_CH), jnp.float32),
    )(partials)


def kernel(x, edge_index, W1, W2):
    h = _mlp(x, W1, W2)
    n_pad = N_CHUNKS_PAD * CHUNK - N_EDGES
    src = jnp.pad(edge_index[0].astype(jnp.int32), (0, n_pad),
                  constant_values=0).reshape(N_CHUNKS_PAD, CHUNK)
    dst = jnp.pad(edge_index[1].astype(jnp.int32), (0, n_pad),
                  constant_values=DUMP_ROW).reshape(N_CHUNKS_PAD, CHUNK)
    partials = _sc_scatter(h, src, dst)
    return _combine(partials)


# speedup vs baseline: 2.2037x; 2.2037x over previous
import functools

import jax
import jax.numpy as jnp
from jax import lax
from jax.experimental import pallas as pl
from jax.experimental.pallas import tpu as pltpu
from jax.experimental.pallas import tpu_sc as plsc

N_NODES = 10000
N_EDGES = 160000
IN_CH = 256
HIDDEN = 512
OUT_CH = 128

NC = 2
NS = 16
NW = NC * NS
CHUNK = 128
CHUNKS_PER_TILE = 40
N_CHUNKS_PAD = NW * CHUNKS_PER_TILE
ACC_ROWS = N_NODES + 16
DUMP_ROW = N_NODES
ROWS_A = 624
ROWS_LAST = N_NODES - (NS - 1) * ROWS_A


def _mlp_body(x_ref, w1_ref, w2_ref, h_ref):
    t = jnp.dot(x_ref[...], w1_ref[...], preferred_element_type=jnp.float32)
    t = jnp.maximum(t, 0.0)
    h_ref[...] = jnp.dot(t, w2_ref[...], preferred_element_type=jnp.float32)


def _mlp(x, W1, W2):
    R = 1000
    return pl.pallas_call(
        _mlp_body,
        grid=(N_NODES // R,),
        in_specs=[
            pl.BlockSpec((R, IN_CH), lambda i: (i, 0)),
            pl.BlockSpec((IN_CH, HIDDEN), lambda i: (0, 0)),
            pl.BlockSpec((HIDDEN, OUT_CH), lambda i: (0, 0)),
        ],
        out_specs=pl.BlockSpec((R, OUT_CH), lambda i: (i, 0)),
        out_shape=jax.ShapeDtypeStruct((N_NODES, OUT_CH), jnp.float32),
    )(x, W1, W2)


def _sc_body(h_hbm, src_hbm, dst_hbm, out_hbm,
             acc, srcbuf, dstbuf, rows, sem_a):
    cid = lax.axis_index("c")
    sid = lax.axis_index("s")
    wid = sid * NC + cid
    rbase = pl.multiple_of(sid * ROWS_A, 8)

    cbase = pl.multiple_of(wid * CHUNKS_PER_TILE, 8)
    pltpu.sync_copy(src_hbm.at[pl.ds(cbase, CHUNKS_PER_TILE)], srcbuf)
    pltpu.sync_copy(dst_hbm.at[pl.ds(cbase, CHUNKS_PER_TILE)], dstbuf)

    def zrow(i, carry):
        def zcol(j, c2):
            rows[0, i, pl.ds(j * 16, 16)] = jnp.zeros((16,), jnp.float32)
            return c2
        return lax.fori_loop(0, OUT_CH // 16, zcol, carry)
    lax.fori_loop(0, CHUNK, zrow, 0)

    @pl.when(sid < NS - 1)
    def _():
        for k in range(ROWS_A // CHUNK):
            pltpu.sync_copy(rows.at[0], acc.at[pl.ds(rbase + k * CHUNK, CHUNK)])
        tail = ROWS_A % CHUNK
        pltpu.sync_copy(rows.at[0].at[pl.ds(0, tail)],
                        acc.at[pl.ds(rbase + ROWS_A - tail, tail)])

    @pl.when(sid == NS - 1)
    def _():
        for k in range(ROWS_LAST // CHUNK):
            pltpu.sync_copy(rows.at[0], acc.at[pl.ds(rbase + k * CHUNK, CHUNK)])
    plsc.subcore_barrier()

    n_real = 1 + (N_EDGES // CHUNK - 1 - wid * CHUNKS_PER_TILE)

    def step(i, carry):
        @pl.when(i < n_real)
        def _():
            pltpu.async_copy(h_hbm.at[srcbuf.at[i]], rows.at[0], sem_a).wait()
            pltpu.sync_copy(rows.at[0], acc.at[dstbuf.at[i]], add=True)
        return carry
    lax.fori_loop(0, CHUNKS_PER_TILE, step, 0)

    plsc.subcore_barrier()

    @pl.when(sid < NS - 1)
    def _():
        for k in range(ROWS_A // CHUNK):
            sl = pl.ds(rbase + k * CHUNK, CHUNK)
            pltpu.sync_copy(acc.at[sl], rows.at[0])
            pltpu.sync_copy(rows.at[0], out_hbm.at[cid, sl])
        tail = ROWS_A % CHUNK
        sl = pl.ds(rbase + ROWS_A - tail, tail)
        pltpu.sync_copy(acc.at[sl], rows.at[0].at[pl.ds(0, tail)])
        pltpu.sync_copy(rows.at[0].at[pl.ds(0, tail)], out_hbm.at[cid, sl])

    @pl.when(sid == NS - 1)
    def _():
        for k in range(ROWS_LAST // CHUNK):
            sl = pl.ds(rbase + k * CHUNK, CHUNK)
            pltpu.sync_copy(acc.at[sl], rows.at[0])
            pltpu.sync_copy(rows.at[0], out_hbm.at[cid, sl])


def _sc_scatter(h, src2d, dst2d):
    mesh = plsc.VectorSubcoreMesh(core_axis_name="c", subcore_axis_name="s")
    fn = pl.kernel(
        _sc_body,
        out_type=jax.ShapeDtypeStruct((NC, N_NODES, OUT_CH), jnp.float32),
        mesh=mesh,
        scratch_types=[
            pltpu.VMEM_SHARED((ACC_ROWS, OUT_CH), jnp.float32),
            pltpu.VMEM((CHUNKS_PER_TILE, CHUNK), jnp.int32),
            pltpu.VMEM((CHUNKS_PER_TILE, CHUNK), jnp.int32),
            pltpu.VMEM((2, CHUNK, OUT_CH), jnp.float32),
            pltpu.SemaphoreType.DMA,
        ],
    )
    return fn(h, src2d, dst2d)


def _add_body(p_ref, o_ref):
    o_ref[...] = p_ref[0] + p_ref[1]


def _combine(partials):
    R = 2000
    return pl.pallas_call(
        _add_body,
        grid=(N_NODES // R,),
        in_specs=[pl.BlockSpec((NC, R, OUT_CH), lambda i: (0, i, 0))],
        out_specs=pl.BlockSpec((R, OUT_CH), lambda i: (i, 0)),
        out_shape=jax.ShapeDtypeStruct((N_NODES, OUT_CH), jnp.float32),
    )(partials)


def kernel(x, edge_index, W1, W2):
    h = _mlp(x, W1, W2)
    n_pad = N_CHUNKS_PAD * CHUNK - N_EDGES
    src = jnp.pad(edge_index[0].astype(jnp.int32), (0, n_pad),
                  constant_values=0).reshape(N_CHUNKS_PAD, CHUNK)
    dst = jnp.pad(edge_index[1].astype(jnp.int32), (0, n_pad),
                  constant_values=DUMP_ROW).reshape(N_CHUNKS_PAD, CHUNK)
    partials = _sc_scatter(h, src, dst)
    return _combine(partials)


# speedup vs baseline: 2.4238x; 1.0999x over previous
import functools

import jax
import jax.numpy as jnp
from jax import lax
from jax.experimental import pallas as pl
from jax.experimental.pallas import tpu as pltpu
from jax.experimental.pallas import tpu_sc as plsc

N_NODES = 10000
N_EDGES = 160000
IN_CH = 256
HIDDEN = 512
OUT_CH = 128

NC = 2
NS = 16
NW = NC * NS
CHUNK = 128
CHUNKS_PER_TILE = 40
N_CHUNKS_PAD = NW * CHUNKS_PER_TILE
ACC_ROWS = N_NODES + 16
DUMP_ROW = N_NODES
ROWS_A = 624
ROWS_LAST = N_NODES - (NS - 1) * ROWS_A


def _mlp_body(x_ref, w1_ref, w2_ref, h_ref):
    t = jnp.dot(x_ref[...], w1_ref[...], preferred_element_type=jnp.float32)
    t = jnp.maximum(t, 0.0)
    h_ref[...] = jnp.dot(t, w2_ref[...], preferred_element_type=jnp.float32)


def _mlp(x, W1, W2):
    R = 1000
    return pl.pallas_call(
        _mlp_body,
        grid=(N_NODES // R,),
        in_specs=[
            pl.BlockSpec((R, IN_CH), lambda i: (i, 0)),
            pl.BlockSpec((IN_CH, HIDDEN), lambda i: (0, 0)),
            pl.BlockSpec((HIDDEN, OUT_CH), lambda i: (0, 0)),
        ],
        out_specs=pl.BlockSpec((R, OUT_CH), lambda i: (i, 0)),
        out_shape=jax.ShapeDtypeStruct((N_NODES, OUT_CH), jnp.float32),
    )(x, W1, W2)


def _sc_body(h_hbm, src_hbm, dst_hbm, out_hbm,
             acc, srcbuf, dstbuf, rows, sem_a, sem_b):
    cid = lax.axis_index("c")
    sid = lax.axis_index("s")
    wid = sid * NC + cid
    rbase = pl.multiple_of(sid * ROWS_A, 8)

    cbase = pl.multiple_of(wid * CHUNKS_PER_TILE, 8)
    pltpu.sync_copy(src_hbm.at[pl.ds(cbase, CHUNKS_PER_TILE)], srcbuf)
    pltpu.sync_copy(dst_hbm.at[pl.ds(cbase, CHUNKS_PER_TILE)], dstbuf)

    def zrow(i, carry):
        def zcol(j, c2):
            rows[0, i, pl.ds(j * 16, 16)] = jnp.zeros((16,), jnp.float32)
            return c2
        return lax.fori_loop(0, OUT_CH // 16, zcol, carry)
    lax.fori_loop(0, CHUNK, zrow, 0)

    @pl.when(sid < NS - 1)
    def _():
        for k in range(ROWS_A // CHUNK):
            pltpu.sync_copy(rows.at[0], acc.at[pl.ds(rbase + k * CHUNK, CHUNK)])
        tail = ROWS_A % CHUNK
        pltpu.sync_copy(rows.at[0].at[pl.ds(0, tail)],
                        acc.at[pl.ds(rbase + ROWS_A - tail, tail)])

    @pl.when(sid == NS - 1)
    def _():
        for k in range(ROWS_LAST // CHUNK):
            pltpu.sync_copy(rows.at[0], acc.at[pl.ds(rbase + k * CHUNK, CHUNK)])
    plsc.subcore_barrier()

    n_real = 1 + (N_EDGES // CHUNK - 1 - wid * CHUNKS_PER_TILE)

    def step(j, carry):
        c0 = 2 * j

        @pl.when(c0 < n_real)
        def _():
            da = pltpu.async_copy(h_hbm.at[srcbuf.at[c0]], rows.at[0], sem_a)
            db = pltpu.async_copy(h_hbm.at[srcbuf.at[c0 + 1]], rows.at[1],
                                  sem_b)
            da.wait()
            pltpu.sync_copy(rows.at[0], acc.at[dstbuf.at[c0]], add=True)
            db.wait()
            pltpu.sync_copy(rows.at[1], acc.at[dstbuf.at[c0 + 1]], add=True)
        return carry
    lax.fori_loop(0, CHUNKS_PER_TILE // 2, step, 0)

    plsc.subcore_barrier()

    @pl.when(sid < NS - 1)
    def _():
        for k in range(ROWS_A // CHUNK):
            sl = pl.ds(rbase + k * CHUNK, CHUNK)
            pltpu.sync_copy(acc.at[sl], rows.at[0])
            pltpu.sync_copy(rows.at[0], out_hbm.at[cid, sl])
        tail = ROWS_A % CHUNK
        sl = pl.ds(rbase + ROWS_A - tail, tail)
        pltpu.sync_copy(acc.at[sl], rows.at[0].at[pl.ds(0, tail)])
        pltpu.sync_copy(rows.at[0].at[pl.ds(0, tail)], out_hbm.at[cid, sl])

    @pl.when(sid == NS - 1)
    def _():
        for k in range(ROWS_LAST // CHUNK):
            sl = pl.ds(rbase + k * CHUNK, CHUNK)
            pltpu.sync_copy(acc.at[sl], rows.at[0])
            pltpu.sync_copy(rows.at[0], out_hbm.at[cid, sl])


def _sc_scatter(h, src2d, dst2d):
    mesh = plsc.VectorSubcoreMesh(core_axis_name="c", subcore_axis_name="s")
    fn = pl.kernel(
        _sc_body,
        out_type=jax.ShapeDtypeStruct((NC, N_NODES, OUT_CH), jnp.float32),
        mesh=mesh,
        scratch_types=[
            pltpu.VMEM_SHARED((ACC_ROWS, OUT_CH), jnp.float32),
            pltpu.VMEM((CHUNKS_PER_TILE, CHUNK), jnp.int32),
            pltpu.VMEM((CHUNKS_PER_TILE, CHUNK), jnp.int32),
            pltpu.VMEM((2, CHUNK, OUT_CH), jnp.float32),
            pltpu.SemaphoreType.DMA,
            pltpu.SemaphoreType.DMA,
        ],
    )
    return fn(h, src2d, dst2d)


def _add_body(p_ref, o_ref):
    o_ref[...] = p_ref[0] + p_ref[1]


def _combine(partials):
    R = 2000
    return pl.pallas_call(
        _add_body,
        grid=(N_NODES // R,),
        in_specs=[pl.BlockSpec((NC, R, OUT_CH), lambda i: (0, i, 0))],
        out_specs=pl.BlockSpec((R, OUT_CH), lambda i: (i, 0)),
        out_shape=jax.ShapeDtypeStruct((N_NODES, OUT_CH), jnp.float32),
    )(partials)


def kernel(x, edge_index, W1, W2):
    h = _mlp(x, W1, W2)
    n_pad = N_CHUNKS_PAD * CHUNK - N_EDGES
    src = jnp.pad(edge_index[0].astype(jnp.int32), (0, n_pad),
                  constant_values=0).reshape(N_CHUNKS_PAD, CHUNK)
    dst = jnp.pad(edge_index[1].astype(jnp.int32), (0, n_pad),
                  constant_values=DUMP_ROW).reshape(N_CHUNKS_PAD, CHUNK)
    partials = _sc_scatter(h, src, dst)
    return _combine(partials)


# speedup vs baseline: 2.4249x; 1.0005x over previous
import functools

import jax
import jax.numpy as jnp
from jax import lax
from jax.experimental import pallas as pl
from jax.experimental.pallas import tpu as pltpu
from jax.experimental.pallas import tpu_sc as plsc

N_NODES = 10000
N_EDGES = 160000
IN_CH = 256
HIDDEN = 512
OUT_CH = 128

NC = 2
NS = 16
NW = NC * NS
CHUNK = 128
CHUNKS_PER_TILE = 40
N_CHUNKS_PAD = NW * CHUNKS_PER_TILE
ACC_ROWS = N_NODES + 16
DUMP_ROW = N_NODES
ROWS_A = 624
ROWS_LAST = N_NODES - (NS - 1) * ROWS_A


def _mlp_body(x_ref, w1_ref, w2_ref, h_ref):
    t = jnp.dot(x_ref[...].astype(jnp.bfloat16),
                w1_ref[...].astype(jnp.bfloat16),
                preferred_element_type=jnp.float32)
    t = jnp.maximum(t, 0.0)
    h_ref[...] = jnp.dot(t.astype(jnp.bfloat16),
                         w2_ref[...].astype(jnp.bfloat16),
                         preferred_element_type=jnp.float32)


def _mlp(x, W1, W2):
    R = 1000
    return pl.pallas_call(
        _mlp_body,
        grid=(N_NODES // R,),
        in_specs=[
            pl.BlockSpec((R, IN_CH), lambda i: (i, 0)),
            pl.BlockSpec((IN_CH, HIDDEN), lambda i: (0, 0)),
            pl.BlockSpec((HIDDEN, OUT_CH), lambda i: (0, 0)),
        ],
        out_specs=pl.BlockSpec((R, OUT_CH), lambda i: (i, 0)),
        out_shape=jax.ShapeDtypeStruct((N_NODES, OUT_CH), jnp.float32),
    )(x, W1, W2)


def _sc_body(h_hbm, src_hbm, dst_hbm, out_hbm,
             acc, srcbuf, dstbuf, rows, sem_a, sem_b):
    cid = lax.axis_index("c")
    sid = lax.axis_index("s")
    wid = sid * NC + cid
    rbase = pl.multiple_of(sid * ROWS_A, 8)

    cbase = pl.multiple_of(wid * CHUNKS_PER_TILE, 8)
    pltpu.sync_copy(src_hbm.at[pl.ds(cbase, CHUNKS_PER_TILE)], srcbuf)
    pltpu.sync_copy(dst_hbm.at[pl.ds(cbase, CHUNKS_PER_TILE)], dstbuf)

    def zrow(i, carry):
        def zcol(j, c2):
            rows[0, i, pl.ds(j * 16, 16)] = jnp.zeros((16,), jnp.float32)
            return c2
        return lax.fori_loop(0, OUT_CH // 16, zcol, carry)
    lax.fori_loop(0, CHUNK, zrow, 0)

    @pl.when(sid < NS - 1)
    def _():
        for k in range(ROWS_A // CHUNK):
            pltpu.sync_copy(rows.at[0], acc.at[pl.ds(rbase + k * CHUNK, CHUNK)])
        tail = ROWS_A % CHUNK
        pltpu.sync_copy(rows.at[0].at[pl.ds(0, tail)],
                        acc.at[pl.ds(rbase + ROWS_A - tail, tail)])

    @pl.when(sid == NS - 1)
    def _():
        for k in range(ROWS_LAST // CHUNK):
            pltpu.sync_copy(rows.at[0], acc.at[pl.ds(rbase + k * CHUNK, CHUNK)])
    plsc.subcore_barrier()

    n_real = 1 + (N_EDGES // CHUNK - 1 - wid * CHUNKS_PER_TILE)

    def step(j, carry):
        c0 = 2 * j

        @pl.when(c0 < n_real)
        def _():
            da = pltpu.async_copy(h_hbm.at[srcbuf.at[c0]], rows.at[0], sem_a)
            db = pltpu.async_copy(h_hbm.at[srcbuf.at[c0 + 1]], rows.at[1],
                                  sem_b)
            da.wait()
            pltpu.sync_copy(rows.at[0], acc.at[dstbuf.at[c0]], add=True)
            db.wait()
            pltpu.sync_copy(rows.at[1], acc.at[dstbuf.at[c0 + 1]], add=True)
        return carry
    lax.fori_loop(0, CHUNKS_PER_TILE // 2, step, 0)

    plsc.subcore_barrier()

    @pl.when(sid < NS - 1)
    def _():
        for k in range(ROWS_A // CHUNK):
            sl = pl.ds(rbase + k * CHUNK, CHUNK)
            pltpu.sync_copy(acc.at[sl], rows.at[0])
            pltpu.sync_copy(rows.at[0], out_hbm.at[cid, sl])
        tail = ROWS_A % CHUNK
        sl = pl.ds(rbase + ROWS_A - tail, tail)
        pltpu.sync_copy(acc.at[sl], rows.at[0].at[pl.ds(0, tail)])
        pltpu.sync_copy(rows.at[0].at[pl.ds(0, tail)], out_hbm.at[cid, sl])

    @pl.when(sid == NS - 1)
    def _():
        for k in range(ROWS_LAST // CHUNK):
            sl = pl.ds(rbase + k * CHUNK, CHUNK)
            pltpu.sync_copy(acc.at[sl], rows.at[0])
            pltpu.sync_copy(rows.at[0], out_hbm.at[cid, sl])


def _sc_scatter(h, src2d, dst2d):
    mesh = plsc.VectorSubcoreMesh(core_axis_name="c", subcore_axis_name="s")
    fn = pl.kernel(
        _sc_body,
        out_type=jax.ShapeDtypeStruct((NC, N_NODES, OUT_CH), jnp.float32),
        mesh=mesh,
        scratch_types=[
            pltpu.VMEM_SHARED((ACC_ROWS, OUT_CH), jnp.float32),
            pltpu.VMEM((CHUNKS_PER_TILE, CHUNK), jnp.int32),
            pltpu.VMEM((CHUNKS_PER_TILE, CHUNK), jnp.int32),
            pltpu.VMEM((2, CHUNK, OUT_CH), jnp.float32),
            pltpu.SemaphoreType.DMA,
            pltpu.SemaphoreType.DMA,
        ],
    )
    return fn(h, src2d, dst2d)


def _add_body(p_ref, o_ref):
    o_ref[...] = p_ref[0] + p_ref[1]


def _combine(partials):
    R = 2000
    return pl.pallas_call(
        _add_body,
        grid=(N_NODES // R,),
        in_specs=[pl.BlockSpec((NC, R, OUT_CH), lambda i: (0, i, 0))],
        out_specs=pl.BlockSpec((R, OUT_CH), lambda i: (i, 0)),
        out_shape=jax.ShapeDtypeStruct((N_NODES, OUT_CH), jnp.float32),
    )(partials)


def kernel(x, edge_index, W1, W2):
    h = _mlp(x, W1, W2)
    n_pad = N_CHUNKS_PAD * CHUNK - N_EDGES
    src = jnp.pad(edge_index[0].astype(jnp.int32), (0, n_pad),
                  constant_values=0).reshape(N_CHUNKS_PAD, CHUNK)
    dst = jnp.pad(edge_index[1].astype(jnp.int32), (0, n_pad),
                  constant_values=DUMP_ROW).reshape(N_CHUNKS_PAD, CHUNK)
    partials = _sc_scatter(h, src, dst)
    return _combine(partials)
